# baseline (speedup 1.0000x reference)
import jax
import jax.numpy as jnp
from jax import lax
from jax.experimental import pallas as pl

F_DIM = 128
THRESH = 0.1
N = 1024


def _norm_cols(x, ones_c):
    del ones_c
    ss = jnp.sum(x * x, axis=0, keepdims=True)
    return x * (1.0 / jnp.maximum(jnp.sqrt(ss), 1e-8))


def _dg(a, b, ca, cb):
    return lax.dot_general(a, b, (((ca,), (cb,)), ((), ())),
                           preferred_element_type=jnp.float32)


def _hyper_kernel(xt_ref, xc1_ref, xc2_ref, w1_ref, b1_ref, w2_ref, b2_ref,
                  t_out_ref, c1_out_ref, c2_out_ref):
    w1 = w1_ref[:]
    w2 = w2_ref[:]
    b1 = b1_ref[:]
    b2 = b2_ref[:]

    blocks = {}
    for b in range(2):
        for name, ref in (("t", xt_ref), ("c1", xc1_ref), ("c2", xc2_ref)):
            blocks[(name, b)] = ref[b]

    xn = {k: _dg(w1, v, 0, 0) + b1 for k, v in blocks.items()}

    layout = {0: (("t", 0), ("t", 1), ("c1", 0)),
              1: (("c2", 0), ("c1", 1), ("c2", 1))}
    dest = {0: (t_out_ref.at[0], t_out_ref.at[1], c1_out_ref.at[0]),
            1: (c2_out_ref.at[0], c1_out_ref.at[1], c2_out_ref.at[1])}

    ones_row = jnp.ones((1, N), dtype=jnp.float32)
    ones_c = jnp.ones((1, F_DIM), dtype=jnp.float32)
    for i in range(2):
        that = _norm_cols(blocks[("t", i)], ones_c)
        masks = []
        for half in ("c1", "c2"):
            chat = _norm_cols(blocks[(half, i)], ones_c)
            sim = _dg(that, chat, 0, 0)
            masks.append((sim > THRESH).astype(jnp.float32))
        m1, m2 = masks

        top_k, botA_k, botB_k = layout[i]
        deg_e = 1.0 + _dg(ones_row, m1, 1, 1) + _dg(ones_row, m2, 1, 1)
        xe = (xn[top_k] + _dg(xn[botA_k], m1, 1, 1)
              + _dg(xn[botB_k], m2, 1, 1)) * (1.0 / deg_e)
        xet = _dg(w2, xe, 0, 0) + b2

        d_top, d_botA, d_botB = dest[i]
        d_top[:, :] = xet
        for m, d in ((m1, d_botA), (m2, d_botB)):
            deg_v = _dg(ones_row, m, 1, 0)
            inv_v = 1.0 / jnp.maximum(deg_v, 1.0)
            d[:, :] = _dg(xet, m, 1, 0) * inv_v


def kernel(X_target, X_context1, X_context2, W1, b1, W2, b2):
    B, C, Hh, Ww = X_target.shape
    n = Hh * Ww
    xt = X_target.reshape(B, C, n)
    xc1 = X_context1.reshape(B, C, n)
    xc2 = X_context2.reshape(B, C, n)

    out_sd = jax.ShapeDtypeStruct((B, C, n), jnp.float32)
    t_out, c1_out, c2_out = pl.pallas_call(
        _hyper_kernel,
        out_shape=(out_sd, out_sd, out_sd),
    )(xt, xc1, xc2, W1, b1.reshape(C, 1), W2, b2.reshape(C, 1))

    shp = (B, C, Hh, Ww)
    return (t_out.reshape(shp), c1_out.reshape(shp), c2_out.reshape(shp))


# speedup vs baseline: 1.0418x; 1.0418x over previous
import jax
import jax.numpy as jnp
from jax import lax
from jax.experimental import pallas as pl

F_DIM = 128
THRESH = 0.1
N = 1024


def _norm_cols(x, ones_c):
    del ones_c
    ss = jnp.sum(x * x, axis=0, keepdims=True)
    return x * (1.0 / jnp.maximum(jnp.sqrt(ss), 1e-8))


def _dg(a, b, ca, cb):
    return lax.dot_general(a, b, (((ca,), (cb,)), ((), ())),
                           preferred_element_type=jnp.float32)


def _hyper_kernel(xt_ref, xc1_ref, xc2_ref, w1_ref, b1_ref, w2_ref, b2_ref,
                  t_out_ref, c1_out_ref, c2_out_ref):
    w1 = w1_ref[:]
    w2 = w2_ref[:]
    b1 = b1_ref[:]
    b2 = b2_ref[:]

    blocks = {}
    for b in range(2):
        for name, ref in (("t", xt_ref), ("c1", xc1_ref), ("c2", xc2_ref)):
            blocks[(name, b)] = ref[b]

    xn = {k: _dg(w1, v, 0, 0) + b1 for k, v in blocks.items()}

    layout = {0: (("t", 0), ("t", 1), ("c1", 0)),
              1: (("c2", 0), ("c1", 1), ("c2", 1))}
    dest = {0: (t_out_ref.at[0], t_out_ref.at[1], c1_out_ref.at[0]),
            1: (c2_out_ref.at[0], c1_out_ref.at[1], c2_out_ref.at[1])}

    ones_row = jnp.ones((1, N), dtype=jnp.float32)
    ones_c = jnp.ones((1, F_DIM), dtype=jnp.float32)
    for i in range(2):
        that = _norm_cols(blocks[("t", i)], ones_c)
        masks = []
        for half in ("c1", "c2"):
            chat = _norm_cols(blocks[(half, i)], ones_c)
            sim = _dg(that, chat, 0, 0)
            masks.append((sim > THRESH).astype(jnp.float32))
        m1, m2 = masks

        top_k, botA_k, botB_k = layout[i]
        deg_e = 1.0 + _dg(ones_row, m1, 1, 1) + _dg(ones_row, m2, 1, 1)
        xe = (xn[top_k] + _dg(xn[botA_k], m1, 1, 1)
              + _dg(xn[botB_k], m2, 1, 1)) * (1.0 / deg_e)
        xet = _dg(w2, xe, 0, 0) + b2

        d_top, d_botA, d_botB = dest[i]
        d_top[:, :] = xet
        for m, d in ((m1, d_botA), (m2, d_botB)):
            deg_v = jnp.sum(m, axis=0, keepdims=True)
            inv_v = 1.0 / jnp.maximum(deg_v, 1.0)
            d[:, :] = _dg(xet, m, 1, 0) * inv_v


def kernel(X_target, X_context1, X_context2, W1, b1, W2, b2):
    B, C, Hh, Ww = X_target.shape
    n = Hh * Ww
    xt = X_target.reshape(B, C, n)
    xc1 = X_context1.reshape(B, C, n)
    xc2 = X_context2.reshape(B, C, n)

    out_sd = jax.ShapeDtypeStruct((B, C, n), jnp.float32)
    t_out, c1_out, c2_out = pl.pallas_call(
        _hyper_kernel,
        out_shape=(out_sd, out_sd, out_sd),
    )(xt, xc1, xc2, W1, b1.reshape(C, 1), W2, b2.reshape(C, 1))

    shp = (B, C, Hh, Ww)
    return (t_out.reshape(shp), c1_out.reshape(shp), c2_out.reshape(shp))
